# Initial kernel scaffold; baseline (speedup 1.0000x reference)
import jax
import jax.numpy as jnp
from jax.experimental import pallas as pl


def kernel(z, centers):
    raise NotImplementedError("write your pallas kernel here")


# speedup vs baseline: 3.2984x; 3.2984x over previous
import functools

import jax
import jax.numpy as jnp
from jax.experimental import pallas as pl

B, ZC, H, W = 4, 32, 32, 32
K = 512
N = B * H * W
PIX = H * W
CHUNK = 128
STEPS = N // CHUNK
PCH = PIX // CHUNK
EPS = 1e-10


def _vq_kernel(z_ref, c_ref, qz_ref, idx_ref, prox_ref, cnt_ref, perp_ref):
    i = pl.program_id(0)

    zt = z_ref[0]
    z = zt.T
    c = c_ref[...]
    cT = c.T

    diff = z[:, :, None] - cT[None, :, :]
    d2 = jnp.sum(diff * diff, axis=1)
    d = jnp.sqrt(d2)

    idx = jnp.argmin(d, axis=1).astype(jnp.int32)
    idx_ref[0, 0, :] = idx

    dmin = jnp.min(d, axis=1, keepdims=True)
    e = jnp.exp(dmin - d)
    prox_ref[...] = e / jnp.sum(e, axis=1, keepdims=True)

    onehot = (jax.lax.broadcasted_iota(jnp.int32, (CHUNK, K), 1)
              == idx[:, None]).astype(jnp.float32)
    qz_ref[0] = jax.lax.dot_general(
        c, onehot, (((0,), (1,)), ((), ())),
        preferred_element_type=jnp.float32)

    blk_cnt = jnp.sum(onehot, axis=0)[None, :]

    @pl.when(i == 0)
    def _init():
        cnt_ref[...] = blk_cnt

    @pl.when(i != 0)
    def _acc():
        cnt_ref[...] += blk_cnt

    @pl.when(i == STEPS - 1)
    def _finish():
        avg = cnt_ref[0] * (1.0 / N)
        ent = -jnp.sum(avg * jnp.log(avg + EPS))
        perp_ref[...] = jnp.exp(ent).reshape(1, 1)


@jax.jit
def _vq(z, centers):
    z3 = z.reshape(B, ZC, PIX)
    qz3, idx3, prox, _cnt, perp = pl.pallas_call(
        _vq_kernel,
        grid=(STEPS,),
        in_specs=[
            pl.BlockSpec((1, ZC, CHUNK), lambda i: (i // PCH, 0, i % PCH)),
            pl.BlockSpec((K, ZC), lambda i: (0, 0)),
        ],
        out_specs=[
            pl.BlockSpec((1, ZC, CHUNK), lambda i: (i // PCH, 0, i % PCH)),
            pl.BlockSpec((1, 1, CHUNK), lambda i: (i, 0, 0)),
            pl.BlockSpec((CHUNK, K), lambda i: (i, 0)),
            pl.BlockSpec((1, K), lambda i: (0, 0)),
            pl.BlockSpec((1, 1), lambda i: (0, 0)),
        ],
        out_shape=[
            jax.ShapeDtypeStruct((B, ZC, PIX), jnp.float32),
            jax.ShapeDtypeStruct((STEPS, 1, CHUNK), jnp.int32),
            jax.ShapeDtypeStruct((N, K), jnp.float32),
            jax.ShapeDtypeStruct((1, K), jnp.float32),
            jax.ShapeDtypeStruct((1, 1), jnp.float32),
        ],
    )(z3, centers)
    qz = qz3.reshape(B, ZC, H, W)
    enc_idx = idx3.reshape(N, 1)
    return qz, perp[0, 0], enc_idx, prox


def kernel(z, centers):
    qz, perp, enc_idx, prox = _vq(z, centers)
    return (qz, 0.0, perp, enc_idx, prox)


# speedup vs baseline: 4.7562x; 1.4420x over previous
import jax
import jax.numpy as jnp
from jax.experimental import pallas as pl

B, ZC, H, W = 4, 32, 32, 32
K = 512
N = B * H * W
PIX = H * W
EPS = 1e-10
BIG = 3.4e38


def _vq_kernel(z_ref, c_ref, qz_ref, idx_ref, prox_ref, cnt_ref, perp_ref):
    i = pl.program_id(0)

    zt = z_ref[0]
    z = zt.T
    c = c_ref[...]

    z2 = jnp.sum(z * z, axis=1)
    c2 = jnp.sum(c * c, axis=1)
    zdotc = jax.lax.dot_general(
        z, c, (((1,), (1,)), ((), ())),
        precision=jax.lax.Precision.HIGHEST,
        preferred_element_type=jnp.float32)
    d2a = z2[:, None] - 2.0 * zdotc + c2[None, :]
    da = jnp.sqrt(jnp.maximum(d2a, 0.0))

    dmin = jnp.min(da, axis=1, keepdims=True)
    e = jnp.exp(dmin - da)
    prox_ref[...] = e / jnp.sum(e, axis=1, keepdims=True)

    lane = jax.lax.broadcasted_iota(jnp.int32, (PIX, K), 1)
    i1 = jnp.argmin(da, axis=1).astype(jnp.int32)
    oh1 = (lane == i1[:, None]).astype(jnp.float32)
    da2 = jnp.where(oh1 > 0.0, BIG, da)
    i2 = jnp.argmin(da2, axis=1).astype(jnp.int32)
    oh2 = (lane == i2[:, None]).astype(jnp.float32)

    g1 = jax.lax.dot_general(
        oh1, c, (((1,), (0,)), ((), ())),
        precision=jax.lax.Precision.HIGHEST,
        preferred_element_type=jnp.float32)
    g2 = jax.lax.dot_general(
        oh2, c, (((1,), (0,)), ((), ())),
        precision=jax.lax.Precision.HIGHEST,
        preferred_element_type=jnp.float32)
    r1 = z - g1
    r2 = z - g2
    e1 = jnp.sqrt(jnp.sum(r1 * r1, axis=1))
    e2 = jnp.sqrt(jnp.sum(r2 * r2, axis=1))
    swap = (e2 < e1) | ((e2 == e1) & (i2 < i1))
    idx = jnp.where(swap, i2, i1)
    idx_ref[0, 0, :] = idx

    ohw = jnp.where(swap[:, None], oh2, oh1)
    qz_ref[0] = jax.lax.dot_general(
        c, ohw, (((0,), (1,)), ((), ())),
        precision=jax.lax.Precision.HIGHEST,
        preferred_element_type=jnp.float32)

    blk_cnt = jnp.sum(ohw, axis=0)[None, :]

    @pl.when(i == 0)
    def _init():
        cnt_ref[...] = blk_cnt

    @pl.when(i != 0)
    def _acc():
        cnt_ref[...] += blk_cnt

    @pl.when(i == B - 1)
    def _finish():
        avg = cnt_ref[0] * (1.0 / N)
        ent = -jnp.sum(avg * jnp.log(avg + EPS))
        perp_ref[...] = jnp.exp(ent).reshape(1, 1)


@jax.jit
def _vq(z, centers):
    z3 = z.reshape(B, ZC, PIX)
    qz3, idx3, prox, _cnt, perp = pl.pallas_call(
        _vq_kernel,
        grid=(B,),
        in_specs=[
            pl.BlockSpec((1, ZC, PIX), lambda i: (i, 0, 0)),
            pl.BlockSpec((K, ZC), lambda i: (0, 0)),
        ],
        out_specs=[
            pl.BlockSpec((1, ZC, PIX), lambda i: (i, 0, 0)),
            pl.BlockSpec((1, 1, PIX), lambda i: (i, 0, 0)),
            pl.BlockSpec((PIX, K), lambda i: (i, 0)),
            pl.BlockSpec((1, K), lambda i: (0, 0)),
            pl.BlockSpec((1, 1), lambda i: (0, 0)),
        ],
        out_shape=[
            jax.ShapeDtypeStruct((B, ZC, PIX), jnp.float32),
            jax.ShapeDtypeStruct((B, 1, PIX), jnp.int32),
            jax.ShapeDtypeStruct((N, K), jnp.float32),
            jax.ShapeDtypeStruct((1, K), jnp.float32),
            jax.ShapeDtypeStruct((1, 1), jnp.float32),
        ],
    )(z3, centers)
    qz = qz3.reshape(B, ZC, H, W)
    enc_idx = idx3.reshape(N, 1)
    return qz, perp[0, 0], enc_idx, prox


def kernel(z, centers):
    qz, perp, enc_idx, prox = _vq(z, centers)
    return (qz, 0.0, perp, enc_idx, prox)


# speedup vs baseline: 5.8832x; 1.2370x over previous
import jax
import jax.numpy as jnp
from jax.experimental import pallas as pl

B, ZC, H, W = 4, 32, 32, 32
K = 512
N = B * H * W
PIX = H * W
EPS = 1e-10
BIG = 3.4e38


def _vq_kernel(z_ref, c_ref, qz_ref, idx_ref, prox_ref, cnt_ref, perp_ref):
    i = pl.program_id(0)

    zt = z_ref[0]
    z = zt.T
    c = c_ref[...]

    z2 = jnp.sum(z * z, axis=1)
    c2 = jnp.sum(c * c, axis=1)
    zdotc = jax.lax.dot_general(
        z, c, (((1,), (1,)), ((), ())),
        precision=jax.lax.Precision.HIGHEST,
        preferred_element_type=jnp.float32)
    d2a = z2[:, None] - 2.0 * zdotc + c2[None, :]
    da = jnp.sqrt(jnp.maximum(d2a, 0.0))

    dmin = jnp.min(da, axis=1, keepdims=True)
    e = jnp.exp(dmin - da)
    prox_ref[...] = e / jnp.sum(e, axis=1, keepdims=True)

    lane = jax.lax.broadcasted_iota(jnp.int32, (PIX, K), 1)
    i1 = jnp.argmin(da, axis=1).astype(jnp.int32)
    oh1 = (lane == i1[:, None]).astype(jnp.float32)
    da2 = jnp.where(oh1 > 0.0, BIG, da)
    i2 = jnp.argmin(da2, axis=1).astype(jnp.int32)
    oh2 = (lane == i2[:, None]).astype(jnp.float32)

    c_hi = c.astype(jnp.bfloat16).astype(jnp.float32)
    r = c - c_hi
    c_mid = r.astype(jnp.bfloat16).astype(jnp.float32)
    c_lo = r - c_mid

    def gather_rows(oh):
        def sel(comp):
            return jax.lax.dot_general(
                oh, comp, (((1,), (0,)), ((), ())),
                preferred_element_type=jnp.float32)
        return (sel(c_hi) + sel(c_mid)) + sel(c_lo)

    g1 = gather_rows(oh1)
    g2 = gather_rows(oh2)

    r1 = z - g1
    r2 = z - g2
    e1 = jnp.sqrt(jnp.sum(r1 * r1, axis=1))
    e2 = jnp.sqrt(jnp.sum(r2 * r2, axis=1))
    swap = (e2 < e1) | ((e2 == e1) & (i2 < i1))
    idx = jnp.where(swap, i2, i1)
    idx_ref[0, 0, :] = idx

    ohw = jnp.where(swap[:, None], oh2, oh1)
    qz_ref[0] = ((jax.lax.dot_general(
                     c_hi, ohw, (((0,), (1,)), ((), ())),
                     preferred_element_type=jnp.float32)
                  + jax.lax.dot_general(
                     c_mid, ohw, (((0,), (1,)), ((), ())),
                     preferred_element_type=jnp.float32))
                 + jax.lax.dot_general(
                     c_lo, ohw, (((0,), (1,)), ((), ())),
                     preferred_element_type=jnp.float32))

    blk_cnt = jnp.sum(ohw, axis=0)[None, :]

    @pl.when(i == 0)
    def _init():
        cnt_ref[...] = blk_cnt

    @pl.when(i != 0)
    def _acc():
        cnt_ref[...] += blk_cnt

    @pl.when(i == B - 1)
    def _finish():
        avg = cnt_ref[0] * (1.0 / N)
        ent = -jnp.sum(avg * jnp.log(avg + EPS))
        perp_ref[...] = jnp.exp(ent).reshape(1, 1)


@jax.jit
def _vq(z, centers):
    z3 = z.reshape(B, ZC, PIX)
    qz3, idx3, prox, _cnt, perp = pl.pallas_call(
        _vq_kernel,
        grid=(B,),
        in_specs=[
            pl.BlockSpec((1, ZC, PIX), lambda i: (i, 0, 0)),
            pl.BlockSpec((K, ZC), lambda i: (0, 0)),
        ],
        out_specs=[
            pl.BlockSpec((1, ZC, PIX), lambda i: (i, 0, 0)),
            pl.BlockSpec((1, 1, PIX), lambda i: (i, 0, 0)),
            pl.BlockSpec((PIX, K), lambda i: (i, 0)),
            pl.BlockSpec((1, K), lambda i: (0, 0)),
            pl.BlockSpec((1, 1), lambda i: (0, 0)),
        ],
        out_shape=[
            jax.ShapeDtypeStruct((B, ZC, PIX), jnp.float32),
            jax.ShapeDtypeStruct((B, 1, PIX), jnp.int32),
            jax.ShapeDtypeStruct((N, K), jnp.float32),
            jax.ShapeDtypeStruct((1, K), jnp.float32),
            jax.ShapeDtypeStruct((1, 1), jnp.float32),
        ],
    )(z3, centers)
    qz = qz3.reshape(B, ZC, H, W)
    enc_idx = idx3.reshape(N, 1)
    return qz, perp[0, 0], enc_idx, prox


def kernel(z, centers):
    qz, perp, enc_idx, prox = _vq(z, centers)
    return (qz, 0.0, perp, enc_idx, prox)
